# baseline (speedup 1.0000x reference)
import dataclasses
import functools

import jax
import jax.numpy as jnp
import numpy as np
from jax.experimental import pallas as pl
from jax.experimental.pallas import tpu as pltpu
from jax.experimental.pallas import tpu_sc as plsc

BN_EPS = 1e-5
D = 128
N_NODES = 10000
GATHER_WIN = 128
NW = 32
SNODE = 320
SACC = 336
CWIN = 2048
PAD_PACK = SNODE << 20


def _uv_body(x_ref, wd_ref, wb_ref, u_ref, v_ref):
    xb = x_ref[...]
    u_ref[...] = jnp.dot(xb, wd_ref[...], preferred_element_type=jnp.float32)
    v_ref[...] = jnp.dot(xb, wb_ref[...], preferred_element_type=jnp.float32)


def _uv_project(x, WdT, WbT):
    n = x.shape[0]
    blk = 2000
    return pl.pallas_call(
        _uv_body,
        grid=(n // blk,),
        in_specs=[
            pl.BlockSpec((blk, D), lambda i: (i, 0)),
            pl.BlockSpec((D, D), lambda i: (0, 0)),
            pl.BlockSpec((D, D), lambda i: (0, 0)),
        ],
        out_specs=[
            pl.BlockSpec((blk, D), lambda i: (i, 0)),
            pl.BlockSpec((blk, D), lambda i: (i, 0)),
        ],
        out_shape=[
            jax.ShapeDtypeStruct((n, D), jnp.float32),
            jax.ShapeDtypeStruct((n, D), jnp.float32),
        ],
    )(x, WdT, WbT)


def _sc_gather(U, V, dst_e, src_e, e_pad):
    mesh = plsc.VectorSubcoreMesh(core_axis_name="c", subcore_axis_name="s")

    @functools.partial(
        pl.kernel,
        out_type=[
            jax.ShapeDtypeStruct((e_pad, D), jnp.float32),
            jax.ShapeDtypeStruct((e_pad, D), jnp.float32),
        ],
        mesh=mesh,
    )
    def gather_kernel(u_hbm, v_hbm, di_hbm, si_hbm, gu_hbm, gv_hbm):
        def body(di_v, si_v, gu_v, gv_v):
            def run(sem_u, sem_v):
                cu = pltpu.async_copy(u_hbm.at[di_v.at[0]], gu_v, sem_u)
                cv = pltpu.async_copy(v_hbm.at[si_v.at[0]], gv_v, sem_v)
                cu.wait()
                cv.wait()

            pl.run_scoped(run, pltpu.SemaphoreType.DMA, pltpu.SemaphoreType.DMA)

        pltpu.emit_pipeline(
            body,
            grid=(e_pad // GATHER_WIN,),
            in_specs=[
                pl.BlockSpec((1, GATHER_WIN), index_map=lambda i: (0, i)),
                pl.BlockSpec((1, GATHER_WIN), index_map=lambda i: (0, i)),
            ],
            out_specs=[
                pl.BlockSpec((GATHER_WIN, D), index_map=lambda i: (i, 0)),
                pl.BlockSpec((GATHER_WIN, D), index_map=lambda i: (i, 0)),
            ],
            core_axis_name=("c", "s"),
            dimension_semantics=(pltpu.PARALLEL,),
        )(di_hbm, si_hbm, gu_hbm, gv_hbm)

    return gather_kernel(U, V, dst_e.reshape(1, e_pad), src_e.reshape(1, e_pad))


REGION = CWIN + 128


def _sc_compiler_params():
    cp = pltpu.CompilerParams()
    if "needs_layout_passes" in pltpu.CompilerParams.__dataclass_fields__:
        cp = dataclasses.replace(cp, needs_layout_passes=False)
    return cp


def _sc_compact(dst_e, e_pad):
    mesh = plsc.VectorSubcoreMesh(core_axis_name="c", subcore_axis_name="s")
    nwin = e_pad // CWIN

    @functools.partial(
        pl.kernel,
        out_type=jax.ShapeDtypeStruct((NW, nwin * REGION), jnp.int32),
        mesh=mesh,
        scratch_types=[
            pltpu.VMEM((CWIN,), jnp.int32),
            pltpu.VMEM((REGION,), jnp.int32),
        ],
        compiler_params=_sc_compiler_params(),
    )
    def compact_kernel(dst_hbm, slab_hbm, dwin, lbuf):
        w = jax.lax.axis_index("s") * 2 + jax.lax.axis_index("c")
        lo = w * SNODE
        hi = lo + SNODE
        iota = jax.lax.iota(jnp.int32, 16)

        def window(win, _):
            pltpu.sync_copy(dst_hbm.at[pl.ds(win * CWIN, CWIN)], dwin)

            def vec(k, cnt):
                dv = dwin[pl.ds(k * 16, 16)]
                m = (dv >= lo) & (dv < hi)
                ids = iota + (win * CWIN + k * 16)
                packed = ids | ((dv - lo) << 20)
                plsc.store_compressed(lbuf.at[pl.ds(16 + cnt, 16)], packed,
                                      mask=m)
                return cnt + jnp.sum(m.astype(jnp.int32))

            cnt = jax.lax.fori_loop(0, CWIN // 16, vec, jnp.int32(0))
            lbuf[pl.ds(16 + cnt, 16)] = (iota + win * CWIN) | PAD_PACK
            cnt16 = (cnt + 15) & ~15
            lbuf[pl.ds(0, 16)] = jnp.zeros((16,), jnp.int32) + cnt16
            pltpu.sync_copy(lbuf, slab_hbm.at[w].at[pl.ds(win * REGION, REGION)])
            return 0

        jax.lax.fori_loop(0, nwin, window, 0)

    return compact_kernel(dst_e)


def _sc_segmax(slab, m2, e_pad):
    mesh = plsc.VectorSubcoreMesh(core_axis_name="c", subcore_axis_name="s")
    nwin = e_pad // CWIN

    @functools.partial(
        pl.kernel,
        out_type=jax.ShapeDtypeStruct((NW * SNODE, D), jnp.float32),
        mesh=mesh,
        scratch_types=[
            pltpu.VMEM((SACC, D), jnp.float32),
            pltpu.VMEM((REGION,), jnp.int32),
            pltpu.VMEM((16, D), jnp.float32),
            pltpu.SemaphoreType.DMA,
        ],
        compiler_params=_sc_compiler_params(),
    )
    def segmax_kernel(slab_hbm, m2_hbm, agg_hbm, acc, pwin, rowbuf, sem):
        w = jax.lax.axis_index("s") * 2 + jax.lax.axis_index("c")
        neg = jnp.full((16,), -jnp.inf, jnp.float32)

        def initrow(i, _):
            def initchunk(c, _):
                acc[i, pl.ds(c * 16, 16)] = neg
                return 0

            return jax.lax.fori_loop(0, D // 16, initchunk, 0)

        jax.lax.fori_loop(0, SACC, initrow, 0)

        def window(win, _):
            pltpu.sync_copy(slab_hbm.at[w].at[pl.ds(win * REGION, REGION)],
                            pwin)
            nb = pwin[pl.ds(0, 16)][0] // 16

            def batch(i, _):
                pk = pwin[pl.ds(16 + i * 16, 16)]
                idv = pk & 0xFFFFF
                pltpu.async_copy(m2_hbm.at[idv], rowbuf, sem).wait()

                def edge(j, _):
                    pks = pwin[pl.ds(16 + i * 16 + j, 16)][0]
                    ld = pks >> 20
                    for c in range(D // 16):
                        sl = pl.ds(c * 16, 16)
                        acc[ld, sl] = jnp.maximum(acc[ld, sl], rowbuf[j, sl])
                    return 0

                return jax.lax.fori_loop(0, 16, edge, 0)

            return jax.lax.fori_loop(0, nb, batch, 0)

        jax.lax.fori_loop(0, nwin, window, 0)
        pltpu.sync_copy(acc.at[pl.ds(0, SNODE)],
                        agg_hbm.at[pl.ds(w * SNODE, SNODE)])

    return segmax_kernel(slab, m2)


def _edge_mlp_body(gu_ref, gv_ref, b1_ref, w2_ref, b2_ref, s2_ref, be2_ref,
                   m2_ref):
    z1 = gu_ref[...] + gv_ref[...] + b1_ref[...]
    h = jnp.maximum(z1, 0.0)
    z2 = jnp.dot(h, w2_ref[...], preferred_element_type=jnp.float32)
    z2 = z2 + b2_ref[...]
    m2_ref[...] = jnp.maximum(z2, 0.0) * s2_ref[...] + be2_ref[...]


def _edge_mlp(gU, gV, b1, W2p, b2p, s2, be2):
    e_pad = gU.shape[0]
    blk = 2048
    row = lambda a: a.reshape(1, D)
    return pl.pallas_call(
        _edge_mlp_body,
        grid=(e_pad // blk,),
        in_specs=[
            pl.BlockSpec((blk, D), lambda i: (i, 0)),
            pl.BlockSpec((blk, D), lambda i: (i, 0)),
            pl.BlockSpec((1, D), lambda i: (0, 0)),
            pl.BlockSpec((D, D), lambda i: (0, 0)),
            pl.BlockSpec((1, D), lambda i: (0, 0)),
            pl.BlockSpec((1, D), lambda i: (0, 0)),
            pl.BlockSpec((1, D), lambda i: (0, 0)),
        ],
        out_specs=pl.BlockSpec((blk, D), lambda i: (i, 0)),
        out_shape=jax.ShapeDtypeStruct((e_pad, D), jnp.float32),
    )(gU, gV, row(b1), W2p, row(b2p), row(s2), row(be2))


def _final_mlp_body(a_ref, w_ref, b_ref, s_ref, be_ref, o_ref):
    z = jnp.dot(a_ref[...], w_ref[...], preferred_element_type=jnp.float32)
    z = z + b_ref[...]
    h = jnp.maximum(z, 0.0)
    o_ref[...] = h * s_ref[...] + be_ref[...]


def _final_mlp(agg, W3T, b3, s3, be3):
    n = agg.shape[0]
    blk = 2000
    row = lambda a: a.reshape(1, D)
    return pl.pallas_call(
        _final_mlp_body,
        grid=(n // blk,),
        in_specs=[
            pl.BlockSpec((blk, D), lambda i: (i, 0)),
            pl.BlockSpec((D, D), lambda i: (0, 0)),
            pl.BlockSpec((1, D), lambda i: (0, 0)),
            pl.BlockSpec((1, D), lambda i: (0, 0)),
            pl.BlockSpec((1, D), lambda i: (0, 0)),
        ],
        out_specs=pl.BlockSpec((blk, D), lambda i: (i, 0)),
        out_shape=jax.ShapeDtypeStruct((n, D), jnp.float32),
    )(agg, W3T, row(b3), row(s3), row(be3))


def kernel(x, tpl_edge_index, W1, b1, g1, be1, W2, b2, g2, be2, W3, b3, g3, be3):
    n = x.shape[0]
    inv = 1.0 / np.sqrt(1.0 + BN_EPS)
    s1 = g1 * inv
    s2 = g2 * inv
    s3 = g3 * inv
    W2p = (W2 * s1[None, :]).T
    b2p = b2 + W2 @ be1

    src = tpl_edge_index[0].astype(jnp.int32)
    dst = tpl_edge_index[1].astype(jnp.int32)
    n_edges = src.shape[0]
    loop = jnp.arange(n, dtype=jnp.int32)
    e_real = n_edges + n
    e_pad = ((e_real + GATHER_WIN * NW - 1) // (GATHER_WIN * NW)) * (GATHER_WIN * NW)
    pad = e_pad - e_real
    pad_idx = jnp.arange(pad, dtype=jnp.int32) % jnp.int32(n)
    src_e = jnp.concatenate([src, loop, pad_idx])
    dst_e = jnp.concatenate([dst, loop, pad_idx])

    W1a = W1[:, :D]
    W1b = W1[:, D:]
    U, V = _uv_project(x, (W1a - W1b).T, W1b.T)

    slab = _sc_compact(dst_e, e_pad)
    gU, gV = _sc_gather(U, V, dst_e, src_e, e_pad)
    m2 = _edge_mlp(gU, gV, b1, W2p, b2p, s2, be2)
    agg = _sc_segmax(slab, m2, e_pad)[:n]

    return _final_mlp(agg, W3.T, b3, s3, be3)


# speedup vs baseline: 1.3464x; 1.3464x over previous
import dataclasses
import functools

import jax
import jax.numpy as jnp
import numpy as np
from jax.experimental import pallas as pl
from jax.experimental.pallas import tpu as pltpu
from jax.experimental.pallas import tpu_sc as plsc

BN_EPS = 1e-5
D = 128
N_NODES = 10000
GATHER_WIN = 128
NW = 32
SNODE = 320
SACC = 336
CWIN = 2048
PAD_PACK = SNODE << 20


def _uv_body(x_ref, wd_ref, wb_ref, u_ref, v_ref):
    xb = x_ref[...]
    u_ref[...] = jnp.dot(xb, wd_ref[...], preferred_element_type=jnp.float32)
    v_ref[...] = jnp.dot(xb, wb_ref[...], preferred_element_type=jnp.float32)


def _uv_project(x, WdT, WbT):
    n = x.shape[0]
    blk = 2000
    return pl.pallas_call(
        _uv_body,
        grid=(n // blk,),
        in_specs=[
            pl.BlockSpec((blk, D), lambda i: (i, 0)),
            pl.BlockSpec((D, D), lambda i: (0, 0)),
            pl.BlockSpec((D, D), lambda i: (0, 0)),
        ],
        out_specs=[
            pl.BlockSpec((blk, D), lambda i: (i, 0)),
            pl.BlockSpec((blk, D), lambda i: (i, 0)),
        ],
        out_shape=[
            jax.ShapeDtypeStruct((n, D), jnp.float32),
            jax.ShapeDtypeStruct((n, D), jnp.float32),
        ],
    )(x, WdT, WbT)


def _sc_gather(U, V, dst_e, src_e, e_pad):
    mesh = plsc.VectorSubcoreMesh(core_axis_name="c", subcore_axis_name="s")

    @functools.partial(
        pl.kernel,
        out_type=[
            jax.ShapeDtypeStruct((e_pad, D), jnp.float32),
            jax.ShapeDtypeStruct((e_pad, D), jnp.float32),
        ],
        mesh=mesh,
    )
    def gather_kernel(u_hbm, v_hbm, di_hbm, si_hbm, gu_hbm, gv_hbm):
        def body(di_v, si_v, gu_v, gv_v):
            def run(sem_u, sem_v):
                cu = pltpu.async_copy(u_hbm.at[di_v.at[0]], gu_v, sem_u)
                cv = pltpu.async_copy(v_hbm.at[si_v.at[0]], gv_v, sem_v)
                cu.wait()
                cv.wait()

            pl.run_scoped(run, pltpu.SemaphoreType.DMA, pltpu.SemaphoreType.DMA)

        pltpu.emit_pipeline(
            body,
            grid=(e_pad // GATHER_WIN,),
            in_specs=[
                pl.BlockSpec((1, GATHER_WIN), index_map=lambda i: (0, i)),
                pl.BlockSpec((1, GATHER_WIN), index_map=lambda i: (0, i)),
            ],
            out_specs=[
                pl.BlockSpec((GATHER_WIN, D), index_map=lambda i: (i, 0)),
                pl.BlockSpec((GATHER_WIN, D), index_map=lambda i: (i, 0)),
            ],
            core_axis_name=("c", "s"),
            dimension_semantics=(pltpu.PARALLEL,),
        )(di_hbm, si_hbm, gu_hbm, gv_hbm)

    return gather_kernel(U, V, dst_e.reshape(1, e_pad), src_e.reshape(1, e_pad))


FLUSH = 2048
BATCH = 128


def _sc_compiler_params():
    cp = pltpu.CompilerParams()
    if "needs_layout_passes" in pltpu.CompilerParams.__dataclass_fields__:
        cp = dataclasses.replace(cp, needs_layout_passes=False)
    return cp


def _sc_compact(dst_e, e_pad):
    mesh = plsc.VectorSubcoreMesh(core_axis_name="c", subcore_axis_name="s")
    nwin = e_pad // CWIN
    cap = e_pad + FLUSH + BATCH

    @functools.partial(
        pl.kernel,
        out_type=[
            jax.ShapeDtypeStruct((NW, cap), jnp.int32),
            jax.ShapeDtypeStruct((NW, cap), jnp.int32),
            jax.ShapeDtypeStruct((NW, 128), jnp.int32),
        ],
        mesh=mesh,
        scratch_types=[
            pltpu.VMEM((CWIN,), jnp.int32),
            pltpu.VMEM((2 * FLUSH + BATCH,), jnp.int32),
            pltpu.VMEM((2 * FLUSH + BATCH,), jnp.int32),
            pltpu.VMEM((16,), jnp.int32),
        ],
        compiler_params=_sc_compiler_params(),
    )
    def compact_kernel(dst_hbm, ids_hbm, ldst_hbm, counts_hbm,
                       dwin, libuf, ldbuf, cntv):
        w = jax.lax.axis_index("s") * 2 + jax.lax.axis_index("c")
        lo = w * SNODE
        hi = lo + SNODE
        iota = jax.lax.iota(jnp.int32, 16)

        def flush(nflush, n_words):
            off = nflush * FLUSH
            pltpu.sync_copy(libuf.at[pl.ds(0, FLUSH)],
                            ids_hbm.at[w].at[pl.ds(off, FLUSH)])
            pltpu.sync_copy(ldbuf.at[pl.ds(0, FLUSH)],
                            ldst_hbm.at[w].at[pl.ds(off, FLUSH)])
            def mv(k, _):
                libuf[pl.ds(k * 16, 16)] = libuf[pl.ds(FLUSH + k * 16, 16)]
                ldbuf[pl.ds(k * 16, 16)] = ldbuf[pl.ds(FLUSH + k * 16, 16)]
                return 0

            jax.lax.fori_loop(0, jnp.maximum(n_words - FLUSH + 15, 0) // 16,
                              mv, 0)

        def window(win, carry):
            cnt, nflush = carry
            pltpu.sync_copy(dst_hbm.at[pl.ds(win * CWIN, CWIN)], dwin)

            def vec(k, cnt):
                dv = dwin[pl.ds(k * 16, 16)]
                m = (dv >= lo) & (dv < hi)
                ids = iota + (win * CWIN + k * 16)
                plsc.store_compressed(libuf.at[pl.ds(cnt, 16)], ids, mask=m)
                plsc.store_compressed(ldbuf.at[pl.ds(cnt, 16)], dv - lo,
                                      mask=m)
                return cnt + plsc.all_reduce_population_count(m)[0]

            cnt = jax.lax.fori_loop(0, CWIN // 16, vec, cnt)

            @pl.when(cnt >= FLUSH)
            def do_flush():
                flush(nflush, cnt)
            return (jnp.where(cnt >= FLUSH, cnt - FLUSH, cnt),
                    jnp.where(cnt >= FLUSH, nflush + 1, nflush))

        cnt, nflush = jax.lax.fori_loop(0, nwin, window,
                                        (jnp.int32(0), jnp.int32(0)))
        for p in range(BATCH // 16):
            libuf[pl.ds(cnt + p * 16, 16)] = iota * 8 + p
            ldbuf[pl.ds(cnt + p * 16, 16)] = jnp.zeros((16,), jnp.int32) + SNODE
        cnt128 = (cnt + BATCH - 1) & ~(BATCH - 1)
        total = nflush * FLUSH + cnt128
        flush(nflush, cnt128)
        cntv[...] = jnp.zeros((16,), jnp.int32) + total
        pltpu.sync_copy(cntv, counts_hbm.at[w].at[pl.ds(0, 16)])

    return compact_kernel(dst_e)


def _sc_segmax(ids, ldst, counts, m2, e_pad):
    mesh = plsc.VectorSubcoreMesh(core_axis_name="c", subcore_axis_name="s")

    @functools.partial(
        pl.kernel,
        out_type=jax.ShapeDtypeStruct((NW * SNODE, D), jnp.float32),
        mesh=mesh,
        scratch_types=[
            pltpu.VMEM((SACC, D), jnp.float32),
            pltpu.VMEM((BATCH,), jnp.int32),
            pltpu.VMEM((BATCH + 16,), jnp.int32),
            pltpu.VMEM((BATCH, D), jnp.float32),
            pltpu.VMEM((16,), jnp.int32),
            pltpu.SemaphoreType.DMA,
        ],
        compiler_params=_sc_compiler_params(),
    )
    def segmax_kernel(ids_hbm, ldst_hbm, counts_hbm, m2_hbm, agg_hbm,
                      acc, idbuf, ldbuf, rowbuf, cntv, sem):
        w = jax.lax.axis_index("s") * 2 + jax.lax.axis_index("c")
        neg = jnp.full((16,), -jnp.inf, jnp.float32)

        def initrow(i, _):
            def initchunk(c, _):
                acc[i, pl.ds(c * 16, 16)] = neg
                return 0

            return jax.lax.fori_loop(0, D // 16, initchunk, 0)

        jax.lax.fori_loop(0, SACC, initrow, 0)

        pltpu.sync_copy(counts_hbm.at[w].at[pl.ds(0, 16)], cntv)
        nb = cntv[...][0] // BATCH

        def batch(b, _):
            pltpu.sync_copy(ids_hbm.at[w].at[pl.ds(b * BATCH, BATCH)], idbuf)
            pltpu.sync_copy(ldst_hbm.at[w].at[pl.ds(b * BATCH, BATCH)],
                            ldbuf.at[pl.ds(0, BATCH)])
            pltpu.async_copy(m2_hbm.at[idbuf], rowbuf, sem).wait()

            def edge(j, _):
                ld = ldbuf[pl.ds(j, 16)][0]
                for c in range(D // 16):
                    sl = pl.ds(c * 16, 16)
                    acc[ld, sl] = jnp.maximum(acc[ld, sl], rowbuf[j, sl])
                return 0

            return jax.lax.fori_loop(0, BATCH, edge, 0)

        jax.lax.fori_loop(0, nb, batch, 0)
        pltpu.sync_copy(acc.at[pl.ds(0, SNODE)],
                        agg_hbm.at[pl.ds(w * SNODE, SNODE)])

    return segmax_kernel(ids, ldst, counts, m2)


def _edge_mlp_body(gu_ref, gv_ref, b1_ref, w2_ref, b2_ref, s2_ref, be2_ref,
                   m2_ref):
    z1 = gu_ref[...] + gv_ref[...] + b1_ref[...]
    h = jnp.maximum(z1, 0.0)
    z2 = jnp.dot(h, w2_ref[...], preferred_element_type=jnp.float32)
    z2 = z2 + b2_ref[...]
    m2_ref[...] = jnp.maximum(z2, 0.0) * s2_ref[...] + be2_ref[...]


def _edge_mlp(gU, gV, b1, W2p, b2p, s2, be2):
    e_pad = gU.shape[0]
    blk = 2048
    row = lambda a: a.reshape(1, D)
    return pl.pallas_call(
        _edge_mlp_body,
        grid=(e_pad // blk,),
        in_specs=[
            pl.BlockSpec((blk, D), lambda i: (i, 0)),
            pl.BlockSpec((blk, D), lambda i: (i, 0)),
            pl.BlockSpec((1, D), lambda i: (0, 0)),
            pl.BlockSpec((D, D), lambda i: (0, 0)),
            pl.BlockSpec((1, D), lambda i: (0, 0)),
            pl.BlockSpec((1, D), lambda i: (0, 0)),
            pl.BlockSpec((1, D), lambda i: (0, 0)),
        ],
        out_specs=pl.BlockSpec((blk, D), lambda i: (i, 0)),
        out_shape=jax.ShapeDtypeStruct((e_pad, D), jnp.float32),
    )(gU, gV, row(b1), W2p, row(b2p), row(s2), row(be2))


def _final_mlp_body(a_ref, w_ref, b_ref, s_ref, be_ref, o_ref):
    z = jnp.dot(a_ref[...], w_ref[...], preferred_element_type=jnp.float32)
    z = z + b_ref[...]
    h = jnp.maximum(z, 0.0)
    o_ref[...] = h * s_ref[...] + be_ref[...]


def _final_mlp(agg, W3T, b3, s3, be3):
    n = agg.shape[0]
    blk = 2000
    row = lambda a: a.reshape(1, D)
    return pl.pallas_call(
        _final_mlp_body,
        grid=(n // blk,),
        in_specs=[
            pl.BlockSpec((blk, D), lambda i: (i, 0)),
            pl.BlockSpec((D, D), lambda i: (0, 0)),
            pl.BlockSpec((1, D), lambda i: (0, 0)),
            pl.BlockSpec((1, D), lambda i: (0, 0)),
            pl.BlockSpec((1, D), lambda i: (0, 0)),
        ],
        out_specs=pl.BlockSpec((blk, D), lambda i: (i, 0)),
        out_shape=jax.ShapeDtypeStruct((n, D), jnp.float32),
    )(agg, W3T, row(b3), row(s3), row(be3))


def kernel(x, tpl_edge_index, W1, b1, g1, be1, W2, b2, g2, be2, W3, b3, g3, be3):
    n = x.shape[0]
    inv = 1.0 / np.sqrt(1.0 + BN_EPS)
    s1 = g1 * inv
    s2 = g2 * inv
    s3 = g3 * inv
    W2p = (W2 * s1[None, :]).T
    b2p = b2 + W2 @ be1

    src = tpl_edge_index[0].astype(jnp.int32)
    dst = tpl_edge_index[1].astype(jnp.int32)
    n_edges = src.shape[0]
    loop = jnp.arange(n, dtype=jnp.int32)
    e_real = n_edges + n
    e_pad = ((e_real + GATHER_WIN * NW - 1) // (GATHER_WIN * NW)) * (GATHER_WIN * NW)
    pad = e_pad - e_real
    pad_idx = jnp.arange(pad, dtype=jnp.int32) % jnp.int32(n)
    src_e = jnp.concatenate([src, loop, pad_idx])
    dst_e = jnp.concatenate([dst, loop, pad_idx])

    W1a = W1[:, :D]
    W1b = W1[:, D:]
    U, V = _uv_project(x, (W1a - W1b).T, W1b.T)

    ids, ldst, counts = _sc_compact(dst_e, e_pad)
    gU, gV = _sc_gather(U, V, dst_e, src_e, e_pad)
    m2 = _edge_mlp(gU, gV, b1, W2p, b2p, s2, be2)
    agg = _sc_segmax(ids, ldst, counts, m2, e_pad)[:n]

    return _final_mlp(agg, W3.T, b3, s3, be3)


# speedup vs baseline: 1.3834x; 1.0274x over previous
import dataclasses
import functools

import jax
import jax.numpy as jnp
import numpy as np
from jax.experimental import pallas as pl
from jax.experimental.pallas import tpu as pltpu
from jax.experimental.pallas import tpu_sc as plsc

BN_EPS = 1e-5
D = 128
N_NODES = 10000
GATHER_WIN = 128
NW = 32
SNODE = 320
SACC = 336
CWIN = 2048
PAD_PACK = SNODE << 20


def _uv_body(x_ref, wd_ref, wb_ref, u_ref, v_ref):
    xb = x_ref[...]
    u_ref[...] = jnp.dot(xb, wd_ref[...], preferred_element_type=jnp.float32)
    v_ref[...] = jnp.dot(xb, wb_ref[...], preferred_element_type=jnp.float32)


def _uv_project(x, WdT, WbT):
    n = x.shape[0]
    blk = 2000
    return pl.pallas_call(
        _uv_body,
        grid=(n // blk,),
        in_specs=[
            pl.BlockSpec((blk, D), lambda i: (i, 0)),
            pl.BlockSpec((D, D), lambda i: (0, 0)),
            pl.BlockSpec((D, D), lambda i: (0, 0)),
        ],
        out_specs=[
            pl.BlockSpec((blk, D), lambda i: (i, 0)),
            pl.BlockSpec((blk, D), lambda i: (i, 0)),
        ],
        out_shape=[
            jax.ShapeDtypeStruct((n, D), jnp.float32),
            jax.ShapeDtypeStruct((n, D), jnp.float32),
        ],
    )(x, WdT, WbT)


def _sc_gather(U, V, dst_e, src_e, e_pad):
    mesh = plsc.VectorSubcoreMesh(core_axis_name="c", subcore_axis_name="s")

    @functools.partial(
        pl.kernel,
        out_type=[
            jax.ShapeDtypeStruct((e_pad, D), jnp.float32),
            jax.ShapeDtypeStruct((e_pad, D), jnp.float32),
        ],
        mesh=mesh,
    )
    def gather_kernel(u_hbm, v_hbm, di_hbm, si_hbm, gu_hbm, gv_hbm):
        def body(di_v, si_v, gu_v, gv_v):
            def run(sem_u, sem_v):
                cu = pltpu.async_copy(u_hbm.at[di_v.at[0]], gu_v, sem_u)
                cv = pltpu.async_copy(v_hbm.at[si_v.at[0]], gv_v, sem_v)
                cu.wait()
                cv.wait()

            pl.run_scoped(run, pltpu.SemaphoreType.DMA, pltpu.SemaphoreType.DMA)

        pltpu.emit_pipeline(
            body,
            grid=(e_pad // GATHER_WIN,),
            in_specs=[
                pl.BlockSpec((1, GATHER_WIN), index_map=lambda i: (0, i)),
                pl.BlockSpec((1, GATHER_WIN), index_map=lambda i: (0, i)),
            ],
            out_specs=[
                pl.BlockSpec((GATHER_WIN, D), index_map=lambda i: (i, 0)),
                pl.BlockSpec((GATHER_WIN, D), index_map=lambda i: (i, 0)),
            ],
            core_axis_name=("c", "s"),
            dimension_semantics=(pltpu.PARALLEL,),
        )(di_hbm, si_hbm, gu_hbm, gv_hbm)

    return gather_kernel(U, V, dst_e.reshape(1, e_pad), src_e.reshape(1, e_pad))


FLUSH = 2048
BATCH = 128


def _sc_compiler_params():
    cp = pltpu.CompilerParams()
    if "needs_layout_passes" in pltpu.CompilerParams.__dataclass_fields__:
        cp = dataclasses.replace(cp, needs_layout_passes=False)
    return cp


def _sc_compact(dst_e, e_pad):
    mesh = plsc.VectorSubcoreMesh(core_axis_name="c", subcore_axis_name="s")
    nwin = e_pad // CWIN
    cap = e_pad + FLUSH + BATCH

    @functools.partial(
        pl.kernel,
        out_type=[
            jax.ShapeDtypeStruct((NW, cap), jnp.int32),
            jax.ShapeDtypeStruct((NW, 128), jnp.int32),
        ],
        mesh=mesh,
        scratch_types=[
            pltpu.VMEM((CWIN,), jnp.int32),
            pltpu.VMEM((2 * FLUSH + BATCH,), jnp.int32),
            pltpu.VMEM((16,), jnp.int32),
        ],
        compiler_params=_sc_compiler_params(),
    )
    def compact_kernel(dst_hbm, ids_hbm, counts_hbm, dwin, libuf, cntv):
        w = jax.lax.axis_index("s") * 2 + jax.lax.axis_index("c")
        lo = w * SNODE
        hi = lo + SNODE
        iota = jax.lax.iota(jnp.int32, 16)

        def flush(nflush, n_words):
            off = nflush * FLUSH
            pltpu.sync_copy(libuf.at[pl.ds(0, FLUSH)],
                            ids_hbm.at[w].at[pl.ds(off, FLUSH)])
            def mv(k, _):
                libuf[pl.ds(k * 16, 16)] = libuf[pl.ds(FLUSH + k * 16, 16)]
                return 0

            jax.lax.fori_loop(0, jnp.maximum(n_words - FLUSH + 15, 0) // 16,
                              mv, 0)

        def window(win, carry):
            cnt, nflush = carry
            pltpu.sync_copy(dst_hbm.at[pl.ds(win * CWIN, CWIN)], dwin)

            def vec(k, cnt):
                dv = dwin[pl.ds(k * 16, 16)]
                m = (dv >= lo) & (dv < hi)
                ids = iota + (win * CWIN + k * 16)
                plsc.store_compressed(libuf.at[pl.ds(cnt, 16)], ids, mask=m)
                return cnt + plsc.all_reduce_population_count(m)[0]

            cnt = jax.lax.fori_loop(0, CWIN // 16, vec, cnt)

            @pl.when(cnt >= FLUSH)
            def do_flush():
                flush(nflush, cnt)
            return (jnp.where(cnt >= FLUSH, cnt - FLUSH, cnt),
                    jnp.where(cnt >= FLUSH, nflush + 1, nflush))

        cnt, nflush = jax.lax.fori_loop(0, nwin, window,
                                        (jnp.int32(0), jnp.int32(0)))
        for p in range(BATCH // 16):
            libuf[pl.ds(cnt + p * 16, 16)] = iota * 8 + p
        cnt128 = (cnt + BATCH - 1) & ~(BATCH - 1)
        total = nflush * FLUSH + cnt128
        flush(nflush, cnt128)
        cntv[...] = jnp.zeros((16,), jnp.int32) + total
        pltpu.sync_copy(cntv, counts_hbm.at[w].at[pl.ds(0, 16)])

    return compact_kernel(dst_e)


def _sc_segmax(ids, dst_e, counts, m2, e_pad):
    mesh = plsc.VectorSubcoreMesh(core_axis_name="c", subcore_axis_name="s")

    @functools.partial(
        pl.kernel,
        out_type=jax.ShapeDtypeStruct((NW * SNODE, D), jnp.float32),
        mesh=mesh,
        scratch_types=[
            pltpu.VMEM((SACC, D), jnp.float32),
            pltpu.VMEM((BATCH,), jnp.int32),
            pltpu.VMEM((BATCH,), jnp.int32),
            pltpu.VMEM((BATCH + 16,), jnp.int32),
            pltpu.VMEM((BATCH, D), jnp.float32),
            pltpu.VMEM((16,), jnp.int32),
            pltpu.SemaphoreType.DMA,
            pltpu.SemaphoreType.DMA,
        ],
        compiler_params=_sc_compiler_params(),
    )
    def segmax_kernel(ids_hbm, dst_hbm, counts_hbm, m2_hbm, agg_hbm,
                      acc, idbuf, dstbuf, ldbuf, rowbuf, cntv, sem, sem2):
        w = jax.lax.axis_index("s") * 2 + jax.lax.axis_index("c")
        lo = w * SNODE
        hi = lo + SNODE
        neg = jnp.full((16,), -jnp.inf, jnp.float32)

        def initrow(i, _):
            def initchunk(c, _):
                acc[i, pl.ds(c * 16, 16)] = neg
                return 0

            return jax.lax.fori_loop(0, D // 16, initchunk, 0)

        jax.lax.fori_loop(0, SACC, initrow, 0)

        pltpu.sync_copy(counts_hbm.at[w].at[pl.ds(0, 16)], cntv)
        nb = cntv[...][0] // BATCH

        def batch(b, _):
            pltpu.sync_copy(ids_hbm.at[w].at[pl.ds(b * BATCH, BATCH)], idbuf)
            cr = pltpu.async_copy(m2_hbm.at[idbuf], rowbuf, sem)
            cd = pltpu.async_copy(dst_hbm.at[idbuf], dstbuf, sem2)
            cd.wait()
            def mkld(k, _):
                dv = dstbuf[pl.ds(k * 16, 16)]
                inr = (dv >= lo) & (dv < hi)
                ldbuf[pl.ds(k * 16, 16)] = jnp.where(inr, dv - lo, SNODE)
                return 0

            jax.lax.fori_loop(0, BATCH // 16, mkld, 0)
            cr.wait()

            def edge(j, _):
                ld = ldbuf[pl.ds(j, 16)][0]
                for c in range(D // 16):
                    sl = pl.ds(c * 16, 16)
                    acc[ld, sl] = jnp.maximum(acc[ld, sl], rowbuf[j, sl])
                return 0

            return jax.lax.fori_loop(0, BATCH, edge, 0)

        jax.lax.fori_loop(0, nb, batch, 0)
        pltpu.sync_copy(acc.at[pl.ds(0, SNODE)],
                        agg_hbm.at[pl.ds(w * SNODE, SNODE)])

    return segmax_kernel(ids, dst_e, counts, m2)


def _edge_mlp_body(gu_ref, gv_ref, b1_ref, w2_ref, b2_ref, s2_ref, be2_ref,
                   m2_ref):
    z1 = gu_ref[...] + gv_ref[...] + b1_ref[...]
    h = jnp.maximum(z1, 0.0)
    z2 = jnp.dot(h, w2_ref[...], preferred_element_type=jnp.float32)
    z2 = z2 + b2_ref[...]
    m2_ref[...] = jnp.maximum(z2, 0.0) * s2_ref[...] + be2_ref[...]


def _edge_mlp(gU, gV, b1, W2p, b2p, s2, be2):
    e_pad = gU.shape[0]
    blk = 2048
    row = lambda a: a.reshape(1, D)
    return pl.pallas_call(
        _edge_mlp_body,
        grid=(e_pad // blk,),
        in_specs=[
            pl.BlockSpec((blk, D), lambda i: (i, 0)),
            pl.BlockSpec((blk, D), lambda i: (i, 0)),
            pl.BlockSpec((1, D), lambda i: (0, 0)),
            pl.BlockSpec((D, D), lambda i: (0, 0)),
            pl.BlockSpec((1, D), lambda i: (0, 0)),
            pl.BlockSpec((1, D), lambda i: (0, 0)),
            pl.BlockSpec((1, D), lambda i: (0, 0)),
        ],
        out_specs=pl.BlockSpec((blk, D), lambda i: (i, 0)),
        out_shape=jax.ShapeDtypeStruct((e_pad, D), jnp.float32),
    )(gU, gV, row(b1), W2p, row(b2p), row(s2), row(be2))


def _final_mlp_body(a_ref, w_ref, b_ref, s_ref, be_ref, o_ref):
    z = jnp.dot(a_ref[...], w_ref[...], preferred_element_type=jnp.float32)
    z = z + b_ref[...]
    h = jnp.maximum(z, 0.0)
    o_ref[...] = h * s_ref[...] + be_ref[...]


def _final_mlp(agg, W3T, b3, s3, be3):
    n = agg.shape[0]
    blk = 2000
    row = lambda a: a.reshape(1, D)
    return pl.pallas_call(
        _final_mlp_body,
        grid=(n // blk,),
        in_specs=[
            pl.BlockSpec((blk, D), lambda i: (i, 0)),
            pl.BlockSpec((D, D), lambda i: (0, 0)),
            pl.BlockSpec((1, D), lambda i: (0, 0)),
            pl.BlockSpec((1, D), lambda i: (0, 0)),
            pl.BlockSpec((1, D), lambda i: (0, 0)),
        ],
        out_specs=pl.BlockSpec((blk, D), lambda i: (i, 0)),
        out_shape=jax.ShapeDtypeStruct((n, D), jnp.float32),
    )(agg, W3T, row(b3), row(s3), row(be3))


def kernel(x, tpl_edge_index, W1, b1, g1, be1, W2, b2, g2, be2, W3, b3, g3, be3):
    n = x.shape[0]
    inv = 1.0 / np.sqrt(1.0 + BN_EPS)
    s1 = g1 * inv
    s2 = g2 * inv
    s3 = g3 * inv
    W2p = (W2 * s1[None, :]).T
    b2p = b2 + W2 @ be1

    src = tpl_edge_index[0].astype(jnp.int32)
    dst = tpl_edge_index[1].astype(jnp.int32)
    n_edges = src.shape[0]
    loop = jnp.arange(n, dtype=jnp.int32)
    e_real = n_edges + n
    e_pad = ((e_real + GATHER_WIN * NW - 1) // (GATHER_WIN * NW)) * (GATHER_WIN * NW)
    pad = e_pad - e_real
    pad_idx = jnp.arange(pad, dtype=jnp.int32) % jnp.int32(n)
    src_e = jnp.concatenate([src, loop, pad_idx])
    dst_e = jnp.concatenate([dst, loop, pad_idx])

    W1a = W1[:, :D]
    W1b = W1[:, D:]
    U, V = _uv_project(x, (W1a - W1b).T, W1b.T)

    gU, gV = _sc_gather(U, V, dst_e, src_e, e_pad)
    m2 = _edge_mlp(gU, gV, b1, W2p, b2p, s2, be2)
    ids, counts = _sc_compact(dst_e, e_pad)
    agg = _sc_segmax(ids, dst_e, counts, m2, e_pad)[:n]

    return _final_mlp(agg, W3.T, b3, s3, be3)


# speedup vs baseline: 1.4196x; 1.0262x over previous
import dataclasses
import functools

import jax
import jax.numpy as jnp
import numpy as np
from jax.experimental import pallas as pl
from jax.experimental.pallas import tpu as pltpu
from jax.experimental.pallas import tpu_sc as plsc

BN_EPS = 1e-5
D = 128
N_NODES = 10000
GATHER_WIN = 128
NW = 32
SNODE = 320
SACC = 336
CWIN = 2048
PAD_PACK = SNODE << 20


def _uv_body(x_ref, wd_ref, wb_ref, u_ref, v_ref):
    xb = x_ref[...]
    u_ref[...] = jnp.dot(xb, wd_ref[...], preferred_element_type=jnp.float32)
    v_ref[...] = jnp.dot(xb, wb_ref[...], preferred_element_type=jnp.float32)


def _uv_project(x, WdT, WbT):
    n = x.shape[0]
    blk = 2000
    return pl.pallas_call(
        _uv_body,
        grid=(n // blk,),
        in_specs=[
            pl.BlockSpec((blk, D), lambda i: (i, 0)),
            pl.BlockSpec((D, D), lambda i: (0, 0)),
            pl.BlockSpec((D, D), lambda i: (0, 0)),
        ],
        out_specs=[
            pl.BlockSpec((blk, D), lambda i: (i, 0)),
            pl.BlockSpec((blk, D), lambda i: (i, 0)),
        ],
        out_shape=[
            jax.ShapeDtypeStruct((n, D), jnp.float32),
            jax.ShapeDtypeStruct((n, D), jnp.float32),
        ],
    )(x, WdT, WbT)


def _sc_gather(U, V, dst_e, src_e, e_pad):
    mesh = plsc.VectorSubcoreMesh(core_axis_name="c", subcore_axis_name="s")

    @functools.partial(
        pl.kernel,
        out_type=[
            jax.ShapeDtypeStruct((e_pad, D), jnp.float32),
            jax.ShapeDtypeStruct((e_pad, D), jnp.float32),
        ],
        mesh=mesh,
    )
    def gather_kernel(u_hbm, v_hbm, di_hbm, si_hbm, gu_hbm, gv_hbm):
        def body(di_v, si_v, gu_v, gv_v):
            def run(sem_u, sem_v):
                cu = pltpu.async_copy(u_hbm.at[di_v.at[0]], gu_v, sem_u)
                cv = pltpu.async_copy(v_hbm.at[si_v.at[0]], gv_v, sem_v)
                cu.wait()
                cv.wait()

            pl.run_scoped(run, pltpu.SemaphoreType.DMA, pltpu.SemaphoreType.DMA)

        pltpu.emit_pipeline(
            body,
            grid=(e_pad // GATHER_WIN,),
            in_specs=[
                pl.BlockSpec((1, GATHER_WIN), index_map=lambda i: (0, i)),
                pl.BlockSpec((1, GATHER_WIN), index_map=lambda i: (0, i)),
            ],
            out_specs=[
                pl.BlockSpec((GATHER_WIN, D), index_map=lambda i: (i, 0)),
                pl.BlockSpec((GATHER_WIN, D), index_map=lambda i: (i, 0)),
            ],
            core_axis_name=("c", "s"),
            dimension_semantics=(pltpu.PARALLEL,),
        )(di_hbm, si_hbm, gu_hbm, gv_hbm)

    return gather_kernel(U, V, dst_e.reshape(1, e_pad), src_e.reshape(1, e_pad))


FLUSH = 2048
BATCH = 256


def _sc_compiler_params():
    cp = pltpu.CompilerParams()
    if "needs_layout_passes" in pltpu.CompilerParams.__dataclass_fields__:
        cp = dataclasses.replace(cp, needs_layout_passes=False)
    return cp


def _sc_compact(dst_e, e_pad):
    mesh = plsc.VectorSubcoreMesh(core_axis_name="c", subcore_axis_name="s")
    nwin = e_pad // CWIN
    cap = e_pad + FLUSH + BATCH

    @functools.partial(
        pl.kernel,
        out_type=[
            jax.ShapeDtypeStruct((NW, cap), jnp.int32),
            jax.ShapeDtypeStruct((NW, 128), jnp.int32),
        ],
        mesh=mesh,
        scratch_types=[
            pltpu.VMEM((CWIN,), jnp.int32),
            pltpu.VMEM((2 * FLUSH + BATCH,), jnp.int32),
            pltpu.VMEM((16,), jnp.int32),
        ],
        compiler_params=_sc_compiler_params(),
    )
    def compact_kernel(dst_hbm, ids_hbm, counts_hbm, dwin, libuf, cntv):
        w = jax.lax.axis_index("s") * 2 + jax.lax.axis_index("c")
        lo = w * SNODE
        hi = lo + SNODE
        iota = jax.lax.iota(jnp.int32, 16)

        def flush(nflush, n_words):
            off = nflush * FLUSH
            pltpu.sync_copy(libuf.at[pl.ds(0, FLUSH)],
                            ids_hbm.at[w].at[pl.ds(off, FLUSH)])
            def mv(k, _):
                libuf[pl.ds(k * 16, 16)] = libuf[pl.ds(FLUSH + k * 16, 16)]
                return 0

            jax.lax.fori_loop(0, jnp.maximum(n_words - FLUSH + 15, 0) // 16,
                              mv, 0)

        def window(win, carry):
            cnt, nflush = carry
            pltpu.sync_copy(dst_hbm.at[pl.ds(win * CWIN, CWIN)], dwin)

            def vec(k, cnt):
                dv = dwin[pl.ds(k * 16, 16)]
                m = (dv >= lo) & (dv < hi)
                ids = iota + (win * CWIN + k * 16)
                plsc.store_compressed(libuf.at[pl.ds(cnt, 16)], ids, mask=m)
                return cnt + plsc.all_reduce_population_count(m)[0]

            cnt = jax.lax.fori_loop(0, CWIN // 16, vec, cnt)

            @pl.when(cnt >= FLUSH)
            def do_flush():
                flush(nflush, cnt)
            return (jnp.where(cnt >= FLUSH, cnt - FLUSH, cnt),
                    jnp.where(cnt >= FLUSH, nflush + 1, nflush))

        cnt, nflush = jax.lax.fori_loop(0, nwin, window,
                                        (jnp.int32(0), jnp.int32(0)))
        for p in range(BATCH // 16):
            libuf[pl.ds(cnt + p * 16, 16)] = iota * 8 + p
        cnt128 = (cnt + BATCH - 1) & ~(BATCH - 1)
        total = nflush * FLUSH + cnt128
        flush(nflush, cnt128)
        cntv[...] = jnp.zeros((16,), jnp.int32) + total
        pltpu.sync_copy(cntv, counts_hbm.at[w].at[pl.ds(0, 16)])

    return compact_kernel(dst_e)


def _sc_segmax(ids, dst_e, counts, m2, e_pad):
    mesh = plsc.VectorSubcoreMesh(core_axis_name="c", subcore_axis_name="s")

    @functools.partial(
        pl.kernel,
        out_type=jax.ShapeDtypeStruct((NW * SNODE, D), jnp.float32),
        mesh=mesh,
        scratch_types=[
            pltpu.VMEM((SACC, D), jnp.float32),
            pltpu.VMEM((BATCH,), jnp.int32),
            pltpu.VMEM((BATCH,), jnp.int32),
            pltpu.VMEM((BATCH + 16,), jnp.int32),
            pltpu.VMEM((BATCH, D), jnp.float32),
            pltpu.VMEM((16,), jnp.int32),
            pltpu.SemaphoreType.DMA,
            pltpu.SemaphoreType.DMA,
        ],
        compiler_params=_sc_compiler_params(),
    )
    def segmax_kernel(ids_hbm, dst_hbm, counts_hbm, m2_hbm, agg_hbm,
                      acc, idbuf, dstbuf, ldbuf, rowbuf, cntv, sem, sem2):
        w = jax.lax.axis_index("s") * 2 + jax.lax.axis_index("c")
        lo = w * SNODE
        hi = lo + SNODE
        neg = jnp.full((16,), -jnp.inf, jnp.float32)

        def initrow(i, _):
            def initchunk(c, _):
                acc[i, pl.ds(c * 16, 16)] = neg
                return 0

            return jax.lax.fori_loop(0, D // 16, initchunk, 0)

        jax.lax.fori_loop(0, SACC, initrow, 0)

        pltpu.sync_copy(counts_hbm.at[w].at[pl.ds(0, 16)], cntv)
        nb = cntv[...][0] // BATCH

        def batch(b, _):
            pltpu.sync_copy(ids_hbm.at[w].at[pl.ds(b * BATCH, BATCH)], idbuf)
            cr = pltpu.async_copy(m2_hbm.at[idbuf], rowbuf, sem)
            cd = pltpu.async_copy(dst_hbm.at[idbuf], dstbuf, sem2)
            cd.wait()
            def mkld(k, _):
                dv = dstbuf[pl.ds(k * 16, 16)]
                inr = (dv >= lo) & (dv < hi)
                ldbuf[pl.ds(k * 16, 16)] = jnp.where(inr, dv - lo, SNODE)
                return 0

            jax.lax.fori_loop(0, BATCH // 16, mkld, 0)
            cr.wait()

            def edge(j, _):
                ld = ldbuf[pl.ds(j, 16)][0]
                for c in range(D // 16):
                    sl = pl.ds(c * 16, 16)
                    acc[ld, sl] = jnp.maximum(acc[ld, sl], rowbuf[j, sl])
                return 0

            return jax.lax.fori_loop(0, BATCH, edge, 0)

        jax.lax.fori_loop(0, nb, batch, 0)
        pltpu.sync_copy(acc.at[pl.ds(0, SNODE)],
                        agg_hbm.at[pl.ds(w * SNODE, SNODE)])

    return segmax_kernel(ids, dst_e, counts, m2)


def _edge_mlp_body(gu_ref, gv_ref, b1_ref, w2_ref, b2_ref, s2_ref, be2_ref,
                   m2_ref):
    z1 = gu_ref[...] + gv_ref[...] + b1_ref[...]
    h = jnp.maximum(z1, 0.0)
    z2 = jnp.dot(h, w2_ref[...], preferred_element_type=jnp.float32)
    z2 = z2 + b2_ref[...]
    m2_ref[...] = jnp.maximum(z2, 0.0) * s2_ref[...] + be2_ref[...]


def _edge_mlp(gU, gV, b1, W2p, b2p, s2, be2):
    e_pad = gU.shape[0]
    blk = 2048
    row = lambda a: a.reshape(1, D)
    return pl.pallas_call(
        _edge_mlp_body,
        grid=(e_pad // blk,),
        in_specs=[
            pl.BlockSpec((blk, D), lambda i: (i, 0)),
            pl.BlockSpec((blk, D), lambda i: (i, 0)),
            pl.BlockSpec((1, D), lambda i: (0, 0)),
            pl.BlockSpec((D, D), lambda i: (0, 0)),
            pl.BlockSpec((1, D), lambda i: (0, 0)),
            pl.BlockSpec((1, D), lambda i: (0, 0)),
            pl.BlockSpec((1, D), lambda i: (0, 0)),
        ],
        out_specs=pl.BlockSpec((blk, D), lambda i: (i, 0)),
        out_shape=jax.ShapeDtypeStruct((e_pad, D), jnp.float32),
    )(gU, gV, row(b1), W2p, row(b2p), row(s2), row(be2))


def _final_mlp_body(a_ref, w_ref, b_ref, s_ref, be_ref, o_ref):
    z = jnp.dot(a_ref[...], w_ref[...], preferred_element_type=jnp.float32)
    z = z + b_ref[...]
    h = jnp.maximum(z, 0.0)
    o_ref[...] = h * s_ref[...] + be_ref[...]


def _final_mlp(agg, W3T, b3, s3, be3):
    n = agg.shape[0]
    blk = 2000
    row = lambda a: a.reshape(1, D)
    return pl.pallas_call(
        _final_mlp_body,
        grid=(n // blk,),
        in_specs=[
            pl.BlockSpec((blk, D), lambda i: (i, 0)),
            pl.BlockSpec((D, D), lambda i: (0, 0)),
            pl.BlockSpec((1, D), lambda i: (0, 0)),
            pl.BlockSpec((1, D), lambda i: (0, 0)),
            pl.BlockSpec((1, D), lambda i: (0, 0)),
        ],
        out_specs=pl.BlockSpec((blk, D), lambda i: (i, 0)),
        out_shape=jax.ShapeDtypeStruct((n, D), jnp.float32),
    )(agg, W3T, row(b3), row(s3), row(be3))


def kernel(x, tpl_edge_index, W1, b1, g1, be1, W2, b2, g2, be2, W3, b3, g3, be3):
    n = x.shape[0]
    inv = 1.0 / np.sqrt(1.0 + BN_EPS)
    s1 = g1 * inv
    s2 = g2 * inv
    s3 = g3 * inv
    W2p = (W2 * s1[None, :]).T
    b2p = b2 + W2 @ be1

    src = tpl_edge_index[0].astype(jnp.int32)
    dst = tpl_edge_index[1].astype(jnp.int32)
    n_edges = src.shape[0]
    loop = jnp.arange(n, dtype=jnp.int32)
    e_real = n_edges + n
    e_pad = ((e_real + GATHER_WIN * NW - 1) // (GATHER_WIN * NW)) * (GATHER_WIN * NW)
    pad = e_pad - e_real
    pad_idx = jnp.arange(pad, dtype=jnp.int32) % jnp.int32(n)
    src_e = jnp.concatenate([src, loop, pad_idx])
    dst_e = jnp.concatenate([dst, loop, pad_idx])

    W1a = W1[:, :D]
    W1b = W1[:, D:]
    U, V = _uv_project(x, (W1a - W1b).T, W1b.T)

    gU, gV = _sc_gather(U, V, dst_e, src_e, e_pad)
    m2 = _edge_mlp(gU, gV, b1, W2p, b2p, s2, be2)
    ids, counts = _sc_compact(dst_e, e_pad)
    agg = _sc_segmax(ids, dst_e, counts, m2, e_pad)[:n]

    return _final_mlp(agg, W3.T, b3, s3, be3)
